# Initial kernel scaffold; baseline (speedup 1.0000x reference)
import jax
import jax.numpy as jnp
from jax.experimental import pallas as pl


def kernel(x, W_old, W_new):
    raise NotImplementedError("write your pallas kernel here")


# speedup vs baseline: 4.4421x; 4.4421x over previous
import functools

import jax
import jax.numpy as jnp
from jax import lax
from jax.experimental import pallas as pl
from jax.experimental.pallas import tpu as pltpu
from jax.experimental.pallas import tpu_sc as plsc

N_OLD = 900000
DIM = 64
N = 16384 * 50
NC = 2
NS = 16
NW = NC * NS
C = N // NW
NB = 128
NBLK = C // NB
LIST_PAD = 160

_mesh = plsc.VectorSubcoreMesh(core_axis_name="c", subcore_axis_name="s")
_params = pltpu.CompilerParams(needs_layout_passes=False, use_tc_tiling_on_sc=False)


def _wid():
    return lax.axis_index("s") * NC + lax.axis_index("c")


@functools.partial(
    pl.kernel,
    out_type=jax.ShapeDtypeStruct((NW * 16,), jnp.int32),
    mesh=_mesh,
    compiler_params=_params,
    scratch_types=[
        pltpu.VMEM((C,), jnp.int32),
        pltpu.VMEM((16,), jnp.int32),
    ],
)
def _count_kernel(x_hbm, counts_hbm, chunk_v, cnt_v):
    w = _wid()
    pltpu.sync_copy(x_hbm.at[pl.ds(w * C, C)], chunk_v)

    def body(i, acc):
        v = chunk_v[pl.ds(i * 16, 16)]
        thr = jnp.full((16,), N_OLD, jnp.int32)
        return acc + lax.shift_right_logical(v - thr, 31)

    acc = lax.fori_loop(0, C // 16, body, jnp.zeros((16,), jnp.int32))
    cnt_v[...] = acc
    pltpu.sync_copy(cnt_v, counts_hbm.at[pl.ds(w * 16, 16)])


@functools.partial(
    pl.kernel,
    out_type=jax.ShapeDtypeStruct((N, DIM), jnp.float32),
    mesh=_mesh,
    compiler_params=_params,
    scratch_types=[
        pltpu.VMEM((C,), jnp.int32),
        pltpu.VMEM((C + LIST_PAD,), jnp.int32),
        pltpu.VMEM((C + LIST_PAD,), jnp.int32),
        pltpu.VMEM((NW * 16,), jnp.int32),
        pltpu.VMEM((NB, DIM), jnp.float32),
        pltpu.SemaphoreType.DMA,
    ],
)
def _main_kernel(x_hbm, wold_hbm, wnew_hbm, counts_hbm, out_hbm,
                 chunk_v, old_v, new_v, counts_v, rows_v, sem):
    w = _wid()
    pltpu.sync_copy(x_hbm.at[pl.ds(w * C, C)], chunk_v)
    pltpu.sync_copy(counts_hbm, counts_v)

    k_total = jnp.int32(0)
    p_old = jnp.int32(0)
    p_new = jnp.int32(0)
    for j in range(NW):
        cj = jnp.sum(counts_v[pl.ds(j * 16, 16)])
        before = jnp.int32(j) < w
        k_total = k_total + cj
        p_old = p_old + jnp.where(before, cj, 0)
        p_new = p_new + jnp.where(before, C - cj, 0)

    thr = jnp.full((16,), N_OLD, jnp.int32)

    def part_body(i, carry):
        o, nf = carry
        v = chunk_v[pl.ds(i * 16, 16)]
        m = v < thr
        plsc.store_compressed(old_v.at[pl.ds(o, 16)], v, mask=m)
        plsc.store_compressed(new_v.at[pl.ds(nf, 16)], v - thr,
                              mask=jnp.logical_not(m))
        c = jnp.sum(lax.shift_right_logical(v - thr, 31))
        return o + c, nf + (16 - c)

    n_old, n_new = lax.fori_loop(0, C // 16, part_body,
                                 (jnp.int32(0), jnp.int32(0)))

    z = jnp.zeros((16,), jnp.int32)
    for t in range(9):
        old_v[pl.ds(n_old + 16 * t, 16)] = z
        new_v[pl.ds(n_new + 16 * t, 16)] = z

    def do_list(buf, n, table, base0):
        def blk_body(i, carry):
            bb = i * NB

            @pl.when(bb < n)
            def _():
                pltpu.async_copy(table.at[buf.at[pl.ds(bb, NB)]],
                                 rows_v, sem).wait()
                rem = n - bb

                @pl.when(rem >= NB)
                def _():
                    pltpu.sync_copy(rows_v,
                                    out_hbm.at[pl.ds(base0 + bb, NB)])

                @pl.when(rem < NB)
                def _():
                    def row_body(r, rc):
                        pltpu.sync_copy(rows_v.at[r],
                                        out_hbm.at[base0 + bb + r])
                        return rc

                    lax.fori_loop(0, rem, row_body, jnp.int32(0))


            return carry

        lax.fori_loop(0, NBLK, blk_body, jnp.int32(0))

    do_list(old_v, n_old, wold_hbm, p_old)
    do_list(new_v, C - n_old, wnew_hbm, k_total + p_new)


def kernel(x, W_old, W_new):
    flat = x.reshape(-1).astype(jnp.int32)
    counts = _count_kernel(flat)
    return _main_kernel(flat, W_old, W_new, counts)


# speedup vs baseline: 4.8045x; 1.0816x over previous
import functools

import jax
import jax.numpy as jnp
from jax import lax
from jax.experimental import pallas as pl
from jax.experimental.pallas import tpu as pltpu
from jax.experimental.pallas import tpu_sc as plsc

N_OLD = 900000
DIM = 64
N = 16384 * 50
NC = 2
NS = 16
NW = NC * NS
C = N // NW
NB = 128
NBLK = C // NB
LIST_PAD = 160

_mesh = plsc.VectorSubcoreMesh(core_axis_name="c", subcore_axis_name="s")
_params = pltpu.CompilerParams(needs_layout_passes=False,
                               use_tc_tiling_on_sc=False)


def _wid():
    return lax.axis_index("s") * NC + lax.axis_index("c")


@functools.partial(
    pl.kernel,
    out_type=jax.ShapeDtypeStruct((NW * 16,), jnp.int32),
    mesh=_mesh,
    compiler_params=_params,
    scratch_types=[
        pltpu.VMEM((C,), jnp.int32),
        pltpu.VMEM((16,), jnp.int32),
    ],
)
def _count_kernel(x_hbm, counts_hbm, chunk_v, cnt_v):
    w = _wid()
    pltpu.sync_copy(x_hbm.at[pl.ds(w * C, C)], chunk_v)

    def body(i, acc):
        v = chunk_v[pl.ds(i * 16, 16)]
        thr = jnp.full((16,), N_OLD, jnp.int32)
        return acc + lax.shift_right_logical(v - thr, 31)

    acc = lax.fori_loop(0, C // 16, body, jnp.zeros((16,), jnp.int32))
    cnt_v[...] = acc
    pltpu.sync_copy(cnt_v, counts_hbm.at[pl.ds(w * 16, 16)])


@functools.partial(
    pl.kernel,
    out_type=jax.ShapeDtypeStruct((N, DIM), jnp.float32),
    mesh=_mesh,
    compiler_params=_params,
    scratch_types=[
        pltpu.VMEM((C,), jnp.int32),
        pltpu.VMEM((C + LIST_PAD,), jnp.int32),
        pltpu.VMEM((C + LIST_PAD,), jnp.int32),
        pltpu.VMEM((NW * 16,), jnp.int32),
        pltpu.VMEM((NB, DIM), jnp.float32),
        pltpu.VMEM((NB, DIM), jnp.float32),
        pltpu.SemaphoreType.DMA,
        pltpu.SemaphoreType.DMA,
    ],
)
def _main_kernel(x_hbm, wold_hbm, wnew_hbm, counts_hbm, out_hbm,
                 chunk_v, old_v, new_v, counts_v, rows_a, rows_b,
                 sem_a, sem_b):
    w = _wid()
    pltpu.sync_copy(x_hbm.at[pl.ds(w * C, C)], chunk_v)
    pltpu.sync_copy(counts_hbm, counts_v)

    k_total = jnp.int32(0)
    p_old = jnp.int32(0)
    p_new = jnp.int32(0)
    for j in range(NW):
        cj = jnp.sum(counts_v[pl.ds(j * 16, 16)])
        before = jnp.int32(j) < w
        k_total = k_total + cj
        p_old = p_old + jnp.where(before, cj, 0)
        p_new = p_new + jnp.where(before, C - cj, 0)

    thr = jnp.full((16,), N_OLD, jnp.int32)

    def part_body(i, carry):
        o, nf = carry
        v = chunk_v[pl.ds(i * 16, 16)]
        m = v < thr
        plsc.store_compressed(old_v.at[pl.ds(o, 16)], v, mask=m)
        plsc.store_compressed(new_v.at[pl.ds(nf, 16)], v - thr,
                              mask=jnp.logical_not(m))
        c = jnp.sum(lax.shift_right_logical(v - thr, 31))
        return o + c, nf + (16 - c)

    n_old, n_new = lax.fori_loop(0, C // 16, part_body,
                                 (jnp.int32(0), jnp.int32(0)))

    z = jnp.zeros((16,), jnp.int32)
    for t in range(9):
        old_v[pl.ds(n_old + 16 * t, 16)] = z
        new_v[pl.ds(n_new + 16 * t, 16)] = z

    def do_list(buf, n, table, base0):
        def gather_into(bb, dst, sem):
            pltpu.async_copy(table.at[buf.at[pl.ds(bb, NB)]], dst, sem)

        def wait_gather(dst, sem):
            pltpu.make_async_copy(out_hbm.at[pl.ds(0, NB)], dst, sem).wait()

        def write_blk(bb, src):
            rem = n - bb

            @pl.when(rem >= NB)
            def _():
                pltpu.sync_copy(src, out_hbm.at[pl.ds(base0 + bb, NB)])

            @pl.when(rem < NB)
            def _():
                off = jnp.int32(0)
                for size in (64, 32, 16, 8, 4, 2, 1):
                    has = (rem & size) != 0

                    @pl.when(has)
                    def _(off=off, size=size):
                        pltpu.sync_copy(
                            src.at[pl.ds(off, size)],
                            out_hbm.at[pl.ds(base0 + bb + off, size)])

                    off = off + jnp.where(has, size, 0)

        @pl.when(n > 0)
        def _():
            gather_into(0, rows_a, sem_a)

        def body(i, carry):
            bb = (2 * i) * NB

            @pl.when(bb < n)
            def _():
                wait_gather(rows_a, sem_a)

            @pl.when(bb + NB < n)
            def _():
                gather_into(bb + NB, rows_b, sem_b)

            @pl.when(bb < n)
            def _():
                write_blk(bb, rows_a)

            @pl.when(bb + 2 * NB < n)
            def _():
                gather_into(bb + 2 * NB, rows_a, sem_a)

            @pl.when(bb + NB < n)
            def _():
                wait_gather(rows_b, sem_b)
                write_blk(bb + NB, rows_b)

            return carry

        lax.fori_loop(0, NBLK // 2, body, jnp.int32(0))

    do_list(old_v, n_old, wold_hbm, p_old)
    do_list(new_v, C - n_old, wnew_hbm, k_total + p_new)


def kernel(x, W_old, W_new):
    flat = x.reshape(-1).astype(jnp.int32)
    counts = _count_kernel(flat)
    return _main_kernel(flat, W_old, W_new, counts)
